# Initial kernel scaffold; baseline (speedup 1.0000x reference)
import functools

import jax
import jax.numpy as jnp
from jax import lax
from jax.experimental import pallas as pl
from jax.experimental.pallas import tpu as pltpu
from jax.experimental.pallas import tpu_sc as plsc

_N_HEADS = 8
_TABLE_SIZE = 100000
_D_EMBED = 32

_info = plsc.get_sparse_core_info()
_NC, _NS, _L = _info.num_cores, _info.num_subcores, _info.num_lanes
_NW = _NC * _NS

_BLK = 2048


def _sc_gather(flat_idx, weight, n_rows):
    b_per_w = n_rows // _NW
    n_blk = b_per_w // _BLK
    mesh = plsc.VectorSubcoreMesh(core_axis_name="c", subcore_axis_name="s")

    @functools.partial(
        pl.kernel,
        mesh=mesh,
        out_type=jax.ShapeDtypeStruct((n_rows, _D_EMBED), jnp.float32),
        scratch_types=[
            pltpu.VMEM((_BLK,), jnp.int32),
            pltpu.VMEM((_BLK, _D_EMBED), jnp.float32),
            pltpu.SemaphoreType.DMA,
        ],
    )
    def k(idx_hbm, w_hbm, out_hbm, idx_v, rows_v, sem):
        wid = lax.axis_index("s") * _NC + lax.axis_index("c")
        base = wid * b_per_w
        off = (lax.iota(jnp.int32, _L) % _N_HEADS) * _TABLE_SIZE

        def blk_body(bi, carry):
            bbase = base + bi * _BLK
            pltpu.sync_copy(idx_hbm.at[pl.ds(bbase, _BLK)], idx_v)

            def add_body(i, c):
                s = pl.ds(i * _L, _L)
                idx_v[s] = idx_v[s] + off
                return c

            lax.fori_loop(0, _BLK // _L, add_body, 0)
            pltpu.async_copy(w_hbm.at[idx_v], rows_v, sem).wait()
            pltpu.sync_copy(rows_v, out_hbm.at[pl.ds(bbase, _BLK)])
            return carry

        lax.fori_loop(0, n_blk, blk_body, 0)

    return k


def kernel(indices, weight):
    B, T, H = indices.shape
    n_rows = B * T * H
    flat_idx = indices.reshape(n_rows).astype(jnp.int32)
    out = _sc_gather(flat_idx, weight, n_rows)(flat_idx, weight)
    return out.reshape(B, T, H, _D_EMBED)


# speedup vs baseline: 2.2001x; 2.2001x over previous
import functools

import jax
import jax.numpy as jnp
from jax import lax
from jax.experimental import pallas as pl
from jax.experimental.pallas import tpu as pltpu
from jax.experimental.pallas import tpu_sc as plsc

_N_HEADS = 8
_TABLE_SIZE = 100000
_D_EMBED = 32

_info = plsc.get_sparse_core_info()
_NC, _NS, _L = _info.num_cores, _info.num_subcores, _info.num_lanes
_NW = _NC * _NS

_BLK = 2048


def _sc_gather(flat_idx, weight, n_rows):
    b_per_w = n_rows // _NW
    n_blk = b_per_w // _BLK
    mesh = plsc.VectorSubcoreMesh(core_axis_name="c", subcore_axis_name="s")

    @functools.partial(
        pl.kernel,
        mesh=mesh,
        compiler_params=pltpu.CompilerParams(use_tc_tiling_on_sc=False),
        out_type=jax.ShapeDtypeStruct((n_rows, _D_EMBED), jnp.float32),
        scratch_types=[
            pltpu.VMEM((_BLK,), jnp.int32),
            pltpu.VMEM((_BLK, _D_EMBED), jnp.float32),
            pltpu.SemaphoreType.DMA,
        ],
    )
    def k(idx_hbm, w_hbm, out_hbm, idx_v, rows_v, sem):
        wid = lax.axis_index("s") * _NC + lax.axis_index("c")
        base = wid * b_per_w
        off = (lax.iota(jnp.int32, _L) % _N_HEADS) * _TABLE_SIZE

        def blk_body(bi, carry):
            bbase = base + bi * _BLK
            pltpu.sync_copy(idx_hbm.at[pl.ds(bbase, _BLK)], idx_v)

            def add_body(i, c):
                s = pl.ds(i * _L, _L)
                idx_v[s] = idx_v[s] + off
                return c

            lax.fori_loop(0, _BLK // _L, add_body, 0)
            pltpu.async_copy(w_hbm.at[idx_v], rows_v, sem).wait()
            pltpu.sync_copy(rows_v, out_hbm.at[pl.ds(bbase, _BLK)])
            return carry

        lax.fori_loop(0, n_blk, blk_body, 0)

    return k


def kernel(indices, weight):
    B, T, H = indices.shape
    n_rows = B * T * H
    flat_idx = indices.reshape(n_rows).astype(jnp.int32)
    out = _sc_gather(flat_idx, weight, n_rows)(flat_idx, weight)
    return out.reshape(B, T, H, _D_EMBED)


# speedup vs baseline: 2.2504x; 1.0229x over previous
import functools

import jax
import jax.numpy as jnp
from jax import lax
from jax.experimental import pallas as pl
from jax.experimental.pallas import tpu as pltpu
from jax.experimental.pallas import tpu_sc as plsc

_N_HEADS = 8
_TABLE_SIZE = 100000
_D_EMBED = 32

_info = plsc.get_sparse_core_info()
_NC, _NS, _L = _info.num_cores, _info.num_subcores, _info.num_lanes
_NW = _NC * _NS

_BLK = 1600
_UNROLL = 4


def _sc_gather(n_rows):
    b_per_w = n_rows // _NW
    n_blk = b_per_w // _BLK
    assert b_per_w % _BLK == 0 and n_blk % 2 == 0
    mesh = plsc.VectorSubcoreMesh(core_axis_name="c", subcore_axis_name="s")

    @functools.partial(
        pl.kernel,
        mesh=mesh,
        compiler_params=pltpu.CompilerParams(use_tc_tiling_on_sc=False),
        out_type=jax.ShapeDtypeStruct((n_rows, _D_EMBED), jnp.float32),
        scratch_types=[
            pltpu.VMEM((_BLK,), jnp.int32),
            pltpu.VMEM((_BLK,), jnp.int32),
            pltpu.VMEM((_BLK, _D_EMBED), jnp.float32),
            pltpu.VMEM((_BLK, _D_EMBED), jnp.float32),
            pltpu.SemaphoreType.DMA,
            pltpu.SemaphoreType.DMA,
            pltpu.SemaphoreType.DMA,
            pltpu.SemaphoreType.DMA,
            pltpu.SemaphoreType.DMA,
            pltpu.SemaphoreType.DMA,
        ],
    )
    def k(idx_hbm, w_hbm, out_hbm, idx_a, idx_b, rows_a, rows_b,
          si_a, si_b, sg_a, sg_b, ss_a, ss_b):
        wid = lax.axis_index("s") * _NC + lax.axis_index("c")
        base = wid * b_per_w
        off = (lax.iota(jnp.int32, _L) % _N_HEADS) * _TABLE_SIZE

        def add_offsets(idx_v):
            def body(i, c):
                for u in range(_UNROLL):
                    s = pl.ds((i * _UNROLL + u) * _L, _L)
                    idx_v[s] = idx_v[s] + off
                return c
            lax.fori_loop(0, _BLK // (_L * _UNROLL), body, 0)

        def start_idx(bi, idx_v, sem):
            pltpu.async_copy(idx_hbm.at[pl.ds(base + bi * _BLK, _BLK)],
                             idx_v, sem)

        def wait_idx(idx_v, sem):
            pltpu.make_async_copy(idx_hbm.at[pl.ds(0, _BLK)], idx_v, sem).wait()

        def start_gather(idx_v, rows_v, sem):
            pltpu.async_copy(w_hbm.at[idx_v], rows_v, sem)

        def wait_gather(rows_v, sem):
            pltpu.make_async_copy(w_hbm.at[pl.ds(0, _BLK)], rows_v, sem).wait()

        def start_store(bi, rows_v, sem):
            pltpu.async_copy(rows_v, out_hbm.at[pl.ds(base + bi * _BLK, _BLK)],
                             sem)

        def wait_store(rows_v, sem):
            pltpu.make_async_copy(rows_v, out_hbm.at[pl.ds(0, _BLK)], sem).wait()

        start_idx(0, idx_a, si_a)
        wait_idx(idx_a, si_a)
        add_offsets(idx_a)
        start_gather(idx_a, rows_a, sg_a)
        start_idx(1, idx_b, si_b)

        def pair_body(i, carry):
            bi = i * 2

            wait_idx(idx_b, si_b)
            add_offsets(idx_b)

            @pl.when(bi > 0)
            def _():
                wait_store(rows_b, ss_b)
            wait_gather(rows_a, sg_a)
            start_store(bi, rows_a, ss_a)
            start_gather(idx_b, rows_b, sg_b)

            @pl.when(bi + 2 < n_blk)
            def _():
                start_idx(bi + 2, idx_a, si_a)

            @pl.when(bi + 2 < n_blk)
            def _():
                wait_idx(idx_a, si_a)
                add_offsets(idx_a)
            wait_store(rows_a, ss_a)
            wait_gather(rows_b, sg_b)
            start_store(bi + 1, rows_b, ss_b)

            @pl.when(bi + 2 < n_blk)
            def _():
                start_gather(idx_a, rows_a, sg_a)

            @pl.when(bi + 3 < n_blk)
            def _():
                start_idx(bi + 3, idx_b, si_b)
            return carry

        lax.fori_loop(0, n_blk // 2, pair_body, 0)
        wait_store(rows_b, ss_b)

    return k


def kernel(indices, weight):
    B, T, H = indices.shape
    n_rows = B * T * H
    flat_idx = indices.reshape(n_rows).astype(jnp.int32)
    out = _sc_gather(n_rows)(flat_idx, weight)
    return out.reshape(B, T, H, _D_EMBED)


# speedup vs baseline: 3.1435x; 1.3969x over previous
import functools

import jax
import jax.numpy as jnp
from jax import lax
from jax.experimental import pallas as pl
from jax.experimental.pallas import tpu as pltpu
from jax.experimental.pallas import tpu_sc as plsc
from jax.experimental.layout import Format, Layout, with_layout_constraint

_N_HEADS = 8
_TABLE_SIZE = 100000
_D_EMBED = 32

_info = plsc.get_sparse_core_info()
_NC, _NS, _L = _info.num_cores, _info.num_subcores, _info.num_lanes
_NW = _NC * _NS

_BLK = 1600
_UNROLL = 4


def _sc_gather(n_rows):
    b_per_w = n_rows // _NW
    n_blk = b_per_w // _BLK
    assert b_per_w % _BLK == 0 and n_blk % 2 == 0
    mesh = plsc.VectorSubcoreMesh(core_axis_name="c", subcore_axis_name="s")

    @functools.partial(
        pl.kernel,
        mesh=mesh,
        compiler_params=pltpu.CompilerParams(use_tc_tiling_on_sc=False),
        out_type=jax.ShapeDtypeStruct((n_rows, _D_EMBED), jnp.float32),
        scratch_types=[
            pltpu.VMEM((_BLK,), jnp.int32),
            pltpu.VMEM((_BLK,), jnp.int32),
            pltpu.VMEM((_BLK, _D_EMBED), jnp.float32),
            pltpu.VMEM((_BLK, _D_EMBED), jnp.float32),
            pltpu.SemaphoreType.DMA,
            pltpu.SemaphoreType.DMA,
            pltpu.SemaphoreType.DMA,
            pltpu.SemaphoreType.DMA,
            pltpu.SemaphoreType.DMA,
            pltpu.SemaphoreType.DMA,
        ],
    )
    def k(idx_hbm, w_hbm, out_hbm, idx_a, idx_b, rows_a, rows_b,
          si_a, si_b, sg_a, sg_b, ss_a, ss_b):
        wid = lax.axis_index("s") * _NC + lax.axis_index("c")
        base = wid * b_per_w
        off = (lax.iota(jnp.int32, _L) % _N_HEADS) * _TABLE_SIZE

        def add_offsets(idx_v):
            def body(i, c):
                for u in range(_UNROLL):
                    s = pl.ds((i * _UNROLL + u) * _L, _L)
                    idx_v[s] = idx_v[s] + off
                return c
            lax.fori_loop(0, _BLK // (_L * _UNROLL), body, 0)

        def start_idx(bi, idx_v, sem):
            pltpu.async_copy(idx_hbm.at[pl.ds(base + bi * _BLK, _BLK)],
                             idx_v, sem)

        def wait_idx(idx_v, sem):
            pltpu.make_async_copy(idx_hbm.at[pl.ds(0, _BLK)], idx_v, sem).wait()

        def start_gather(idx_v, rows_v, sem):
            pltpu.async_copy(w_hbm.at[idx_v], rows_v, sem)

        def wait_gather(rows_v, sem):
            pltpu.make_async_copy(w_hbm.at[pl.ds(0, _BLK)], rows_v, sem).wait()

        def start_store(bi, rows_v, sem):
            pltpu.async_copy(rows_v, out_hbm.at[pl.ds(base + bi * _BLK, _BLK)],
                             sem)

        def wait_store(rows_v, sem):
            pltpu.make_async_copy(rows_v, out_hbm.at[pl.ds(0, _BLK)], sem).wait()

        start_idx(0, idx_a, si_a)
        wait_idx(idx_a, si_a)
        add_offsets(idx_a)
        start_gather(idx_a, rows_a, sg_a)
        start_idx(1, idx_b, si_b)

        def pair_body(i, carry):
            bi = i * 2

            wait_idx(idx_b, si_b)
            add_offsets(idx_b)

            @pl.when(bi > 0)
            def _():
                wait_store(rows_b, ss_b)
            wait_gather(rows_a, sg_a)
            start_store(bi, rows_a, ss_a)
            start_gather(idx_b, rows_b, sg_b)

            @pl.when(bi + 2 < n_blk)
            def _():
                start_idx(bi + 2, idx_a, si_a)

            @pl.when(bi + 2 < n_blk)
            def _():
                wait_idx(idx_a, si_a)
                add_offsets(idx_a)
            wait_store(rows_a, ss_a)
            wait_gather(rows_b, sg_b)
            start_store(bi + 1, rows_b, ss_b)

            @pl.when(bi + 2 < n_blk)
            def _():
                start_gather(idx_a, rows_a, sg_a)

            @pl.when(bi + 3 < n_blk)
            def _():
                start_idx(bi + 3, idx_b, si_b)
            return carry

        lax.fori_loop(0, n_blk // 2, pair_body, 0)
        wait_store(rows_b, ss_b)

    return k


def kernel(indices, weight):
    B, T, H = indices.shape
    n_rows = B * T * H
    def _rm(x):
        return with_layout_constraint(
            x, Layout(major_to_minor=tuple(range(x.ndim))))

    indices = _rm(indices)
    weight = _rm(weight)
    flat_idx = indices.reshape(n_rows).astype(jnp.int32)
    out = _sc_gather(n_rows)(flat_idx, weight)
    return _rm(out.reshape(B, T, H, _D_EMBED))
